# Initial kernel scaffold; baseline (speedup 1.0000x reference)
import jax
import jax.numpy as jnp
from jax.experimental import pallas as pl


def kernel(x, edge_index, img_sizes, W_gcn, b_gcn, W_h, b_h, W_c, b_c):
    raise NotImplementedError("write your pallas kernel here")


# speedup vs baseline: 30.2599x; 30.2599x over previous
import functools

import jax
import jax.numpy as jnp
from jax import lax
from jax.experimental import pallas as pl
from jax.experimental.pallas import tpu as pltpu
from jax.experimental.pallas import tpu_sc as plsc

N = 10000
D = 128
E = 320000
NC, NS = 2, 16
NW = NC * NS
NP = 10240
CH = 128
NJ = 80
G = 8
NGRP = NJ // G
E_PAD = NW * NJ * CH
PERS = NP // NS

_mesh = plsc.VectorSubcoreMesh(core_axis_name="c", subcore_axis_name="s")


@functools.partial(
    pl.kernel,
    out_type=jax.ShapeDtypeStruct((NC, NP), jnp.float32),
    mesh=_mesh,
    scratch_types=[
        pltpu.VMEM((NJ, CH), jnp.int32),
        pltpu.VMEM((CH,), jnp.float32),
        pltpu.VMEM((PERS,), jnp.float32),
        pltpu.VMEM_SHARED((NP,), jnp.float32),
        pltpu.SemaphoreType.DMA,
    ],
)
def _deg_kernel(dst_hbm, out_hbm, idx_v, ones_v, zb_v, deg_sh, sem):
    c = lax.axis_index("c")
    s = lax.axis_index("s")
    w = c * NS + s

    def init_ones(i, _):
        ones_v[pl.ds(i * 16, 16)] = jnp.ones((16,), jnp.float32)
        return 0
    lax.fori_loop(0, CH // 16, init_ones, 0)

    def init_z(i, _):
        zb_v[pl.ds(i * 16, 16)] = jnp.zeros((16,), jnp.float32)
        return 0
    lax.fori_loop(0, PERS // 16, init_z, 0)

    pltpu.sync_copy(zb_v, deg_sh.at[pl.ds(s * PERS, PERS)])
    pltpu.async_copy(dst_hbm.at[w], idx_v, sem).wait()
    plsc.subcore_barrier()

    def body(j, _):
        pltpu.sync_copy(ones_v, deg_sh.at[idx_v.at[j]], add=True)
        return 0
    lax.fori_loop(0, NJ, body, 0)

    plsc.subcore_barrier()
    pltpu.sync_copy(deg_sh.at[pl.ds(s * PERS, PERS)],
                    out_hbm.at[c, pl.ds(s * PERS, PERS)])


@functools.partial(
    pl.kernel,
    out_type=jax.ShapeDtypeStruct((NC, NP, D), jnp.float32),
    mesh=_mesh,
    scratch_types=[
        pltpu.VMEM((2, G, CH), jnp.int32),
        pltpu.VMEM((2, G, CH), jnp.int32),
        pltpu.VMEM((2, CH, D), jnp.float32),
        pltpu.VMEM_SHARED((NP, D), jnp.float32),
        pltpu.SemaphoreType.DMA,
        pltpu.SemaphoreType.DMA,
        pltpu.SemaphoreType.DMA,
    ],
)
def _agg_kernel(y_hbm, src_hbm, dst_hbm, out_hbm,
                si_v, di_v, rows_v, acc_sh, semi, sem0, sem1):
    c = lax.axis_index("c")
    s = lax.axis_index("s")
    w = c * NS + s
    sems = (sem0, sem1)

    def zrow(r, _):
        def zcol(k, _):
            rows_v[0, r, pl.ds(k * 16, 16)] = jnp.zeros((16,), jnp.float32)
            return 0
        lax.fori_loop(0, D // 16, zcol, 0)
        return 0
    lax.fori_loop(0, CH, zrow, 0)

    def zcopy(k, _):
        pltpu.sync_copy(rows_v.at[0], acc_sh.at[pl.ds(s * PERS + k * CH, CH)])
        return 0
    lax.fori_loop(0, PERS // CH, zcopy, 0)

    pltpu.async_copy(src_hbm.at[w, pl.ds(0, G)], si_v.at[0], semi)
    pltpu.async_copy(dst_hbm.at[w, pl.ds(0, G)], di_v.at[0], semi)
    pltpu.make_async_copy(src_hbm.at[w, pl.ds(0, G)], si_v.at[0], semi).wait()
    pltpu.make_async_copy(dst_hbm.at[w, pl.ds(0, G)], di_v.at[0], semi).wait()
    plsc.subcore_barrier()
    pltpu.async_copy(y_hbm.at[si_v.at[0, 0]], rows_v.at[0], sem0)

    def group(g, _):
        gb = lax.rem(g, 2)
        gn = lax.rem(g + 1, 2)

        @pl.when(g + 1 < NGRP)
        def _():
            pltpu.async_copy(src_hbm.at[w, pl.ds((g + 1) * G, G)],
                             si_v.at[gn], semi)
            pltpu.async_copy(dst_hbm.at[w, pl.ds((g + 1) * G, G)],
                             di_v.at[gn], semi)

        for j in range(G):
            b = j % 2
            nb = (j + 1) % 2
            pltpu.make_async_copy(y_hbm.at[si_v.at[gb, j]], rows_v.at[b],
                                  sems[b]).wait()
            if j + 1 < G:
                pltpu.async_copy(y_hbm.at[si_v.at[gb, j + 1]], rows_v.at[nb],
                                 sems[nb])
            else:
                @pl.when(g + 1 < NGRP)
                def _():
                    pltpu.make_async_copy(src_hbm.at[w, pl.ds((g + 1) * G, G)],
                                          si_v.at[gn], semi).wait()
                    pltpu.make_async_copy(dst_hbm.at[w, pl.ds((g + 1) * G, G)],
                                          di_v.at[gn], semi).wait()
                    pltpu.async_copy(y_hbm.at[si_v.at[gn, 0]], rows_v.at[nb],
                                     sems[nb])
            pltpu.sync_copy(rows_v.at[b], acc_sh.at[di_v.at[gb, j]], add=True)
        return 0
    lax.fori_loop(0, NGRP, group, 0)

    plsc.subcore_barrier()
    pltpu.sync_copy(acc_sh.at[pl.ds(s * PERS, PERS)],
                    out_hbm.at[c, pl.ds(s * PERS, PERS)])


BLK1 = 512


def _scale_body(x_ref, degT_ref, w_ref, y_ref):
    deg = degT_ref[...]
    dis = lax.rsqrt(deg[:, 0:1] + deg[:, 1:2] + 1.0)
    xw = jnp.dot(x_ref[...], w_ref[...], preferred_element_type=jnp.float32)
    y_ref[...] = xw * dis


def _scale_call(x_pad, degT, W_gcn):
    return pl.pallas_call(
        _scale_body,
        grid=(NP // BLK1,),
        in_specs=[
            pl.BlockSpec((BLK1, D), lambda i: (i, 0)),
            pl.BlockSpec((BLK1, 2), lambda i: (i, 0)),
            pl.BlockSpec((D, D), lambda i: (0, 0)),
        ],
        out_specs=pl.BlockSpec((BLK1, D), lambda i: (i, 0)),
        out_shape=jax.ShapeDtypeStruct((NP, D), jnp.float32),
    )(x_pad, degT, W_gcn)


BLK2 = 400
H = 512
C = 79


def _mlp_body(x_ref, y_ref, acc_ref, degT_ref, bg_ref, wh_ref, bh_ref,
              wc_ref, bc_ref, h_ref, l_ref):
    deg = degT_ref[...]
    dis = lax.rsqrt(deg[:, 0:1] + deg[:, 1:2] + 1.0)
    agg = (acc_ref[0] + acc_ref[1] + y_ref[...]) * dis + bg_ref[...]
    x1 = jnp.maximum(agg, 0.0)
    cat = jnp.concatenate([x_ref[...], x1], axis=1)
    h = jnp.dot(cat, wh_ref[...], preferred_element_type=jnp.float32)
    h = jnp.maximum(h + bh_ref[...], 0.0)
    h_ref[...] = h
    l_ref[...] = jnp.dot(h, wc_ref[...],
                         preferred_element_type=jnp.float32) + bc_ref[...]


def _mlp_call(x, y_pad, accp, degT, b_gcn, W_h, b_h, W_c, b_c):
    return pl.pallas_call(
        _mlp_body,
        grid=(N // BLK2,),
        in_specs=[
            pl.BlockSpec((BLK2, D), lambda i: (i, 0)),
            pl.BlockSpec((BLK2, D), lambda i: (i, 0)),
            pl.BlockSpec((NC, BLK2, D), lambda i: (0, i, 0)),
            pl.BlockSpec((BLK2, 2), lambda i: (i, 0)),
            pl.BlockSpec((D,), lambda i: (0,)),
            pl.BlockSpec((2 * D, H), lambda i: (0, 0)),
            pl.BlockSpec((H,), lambda i: (0,)),
            pl.BlockSpec((H, C), lambda i: (0, 0)),
            pl.BlockSpec((C,), lambda i: (0,)),
        ],
        out_specs=[
            pl.BlockSpec((BLK2, H), lambda i: (i, 0)),
            pl.BlockSpec((BLK2, C), lambda i: (i, 0)),
        ],
        out_shape=[
            jax.ShapeDtypeStruct((N, H), jnp.float32),
            jax.ShapeDtypeStruct((N, C), jnp.float32),
        ],
    )(x, y_pad, accp, degT, b_gcn, W_h, b_h, W_c, b_c)


def kernel(x, edge_index, img_sizes, W_gcn, b_gcn, W_h, b_h, W_c, b_c):
    src = edge_index[0].astype(jnp.int32)
    dst = edge_index[1].astype(jnp.int32)
    pad = N + (jnp.arange(E_PAD - E, dtype=jnp.int32) % (NP - N))
    src_p = jnp.concatenate([src, pad]).reshape(NW, NJ, CH)
    dst_p = jnp.concatenate([dst, pad]).reshape(NW, NJ, CH)

    degp = _deg_kernel(dst_p)
    degT = jnp.swapaxes(degp, 0, 1)
    x_pad = jnp.pad(x, ((0, NP - N), (0, 0)))
    y_pad = _scale_call(x_pad, degT, W_gcn)
    accp = _agg_kernel(y_pad, src_p, dst_p)
    h, logits = _mlp_call(x, y_pad, accp, degT,
                          b_gcn, W_h, b_h, W_c, b_c)
    return (h, logits)
